# baseline (speedup 1.0000x reference)
import functools

import jax
import jax.numpy as jnp
from jax import lax
from jax.experimental import pallas as pl
from jax.experimental.pallas import tpu as pltpu
from jax.experimental.pallas import tpu_sc as plsc

_N = 2097152
_NW = 32
_RW = _N // _NW
_CH = 1024
_NCH = _RW // _CH


def _splat(ref, idx):
    return plsc.load_gather(ref, [jnp.full((16,), idx, jnp.int32)])


def _sc_body(xf, wt_hbm, tokspl_hbm, b_hbm, wspl_hbm, bspl_hbm, out,
             x_buf, o_buf, fill_buf, wt_buf, tokspl_buf, b_buf,
             wspl_buf, bspl_buf):
    c = lax.axis_index("c")
    s = lax.axis_index("s")
    wid = s * 2 + c

    pltpu.sync_copy(wt_hbm, wt_buf)
    pltpu.sync_copy(tokspl_hbm, tokspl_buf)
    pltpu.sync_copy(b_hbm, b_buf)
    pltpu.sync_copy(wspl_hbm, wspl_buf)
    pltpu.sync_copy(bspl_hbm, bspl_buf)

    iota = lax.iota(jnp.int32, 16)
    iota8 = iota * 8
    iota16 = iota * 16

    b_v = b_buf[pl.ds(0, 16)]
    wt_k = [wt_buf[pl.ds(16 * k, 16)] for k in range(8)]

    crow = b_v
    for k in range(8):
        crow = crow + tokspl_buf[pl.ds(16 * k, 16)] * wt_k[k]

    def fill_body(r, carry):
        fill_buf[pl.ds(r * 16, 16)] = crow
        return carry

    lax.fori_loop(0, _CH, fill_body, 0)

    xbase = wid * (_RW * 8)
    obase = wid * (_RW * 16)

    def chunk_body(ci, carry):
        xoff = xbase + ci * (_CH * 8)
        ooff = obase + ci * (_CH * 16)
        pltpu.sync_copy(xf.at[1, pl.ds(xoff, _CH * 8)], x_buf)
        for q in range(4):
            js = [4 * q + jj for jj in range(4)]
            wsp = [[wspl_buf[pl.ds((16 * k + j) * 16, 16)] for k in range(8)]
                   for j in js]
            bsp = [bspl_buf[pl.ds(16 * j, 16)] for j in js]

            def group_body(g, carry):
                xg = g * 128
                og = g * 256
                xv = [plsc.load_gather(x_buf, [iota8 + (xg + k)])
                      for k in range(8)]
                for jj in range(4):
                    acc = bsp[jj]
                    for k in range(8):
                        acc = acc + xv[k] * wsp[jj][k]
                    plsc.store_scatter(o_buf, [iota16 + (og + js[jj])], acc)
                return carry

            lax.fori_loop(0, _CH // 16, group_body, 0)
        pltpu.sync_copy(o_buf, out.at[1, pl.ds(ooff, _CH * 16)])
        pltpu.sync_copy(fill_buf, out.at[0, pl.ds(ooff, _CH * 16)])
        return carry

    lax.fori_loop(0, _NCH, chunk_body, 0)


@functools.partial(jax.jit, static_argnames=())
def kernel(x, token, W, b):
    xf = x.reshape(2, _N * 8)
    wt = W.T.reshape(128)
    tokspl = jnp.repeat(token, 16)
    b128 = jnp.concatenate([b, jnp.zeros((112,), jnp.float32)])
    wspl = jnp.repeat(wt, 16)
    bspl = jnp.repeat(b, 16)
    mesh = plsc.VectorSubcoreMesh(core_axis_name="c", subcore_axis_name="s")
    out = pl.kernel(
        _sc_body,
        out_type=jax.ShapeDtypeStruct((2, _N * 16), jnp.float32),
        mesh=mesh,
        compiler_params=pltpu.CompilerParams(needs_layout_passes=False),
        scratch_types=[
            pltpu.VMEM((_CH * 8,), jnp.float32),
            pltpu.VMEM((_CH * 16,), jnp.float32),
            pltpu.VMEM((_CH * 16,), jnp.float32),
            pltpu.VMEM((128,), jnp.float32),
            pltpu.VMEM((128,), jnp.float32),
            pltpu.VMEM((128,), jnp.float32),
            pltpu.VMEM((2048,), jnp.float32),
            pltpu.VMEM((256,), jnp.float32),
        ],
    )(xf, wt, tokspl, b128, wspl, bspl)
    return out.reshape(2, _N, 16)


# speedup vs baseline: 2.8024x; 2.8024x over previous
import jax
import jax.numpy as jnp
from jax.experimental import pallas as pl


_N = 2097152
_NF = _N // 16
_BG = 4096
_NBLK = _NF // _BG


def _body(tokbig_ref, wbig_ref, bbig_ref, x_ref, o_ref):
    wbig = wbig_ref[...]
    bbig = bbig_ref[...]
    row0 = jnp.dot(tokbig_ref[...], wbig,
                   preferred_element_type=jnp.float32) + bbig
    y1 = jnp.dot(x_ref[0], wbig,
                 preferred_element_type=jnp.float32) + bbig
    o_ref[0] = jnp.broadcast_to(row0, y1.shape)
    o_ref[1] = y1


def kernel(x, token, W, b):
    xv = x.reshape(2, _NF, 128)
    wt = W.T
    wbig = jnp.kron(jnp.eye(16, dtype=jnp.float32), wt)
    bbig = jnp.tile(b, 16).reshape(1, 256)
    tokbig = jnp.tile(token, 16).reshape(1, 128)
    out = pl.pallas_call(
        _body,
        grid=(_NBLK,),
        in_specs=[
            pl.BlockSpec((1, 128), lambda i: (0, 0)),
            pl.BlockSpec((128, 256), lambda i: (0, 0)),
            pl.BlockSpec((1, 256), lambda i: (0, 0)),
            pl.BlockSpec((1, _BG, 128), lambda i: (1, i, 0)),
        ],
        out_specs=pl.BlockSpec((2, _BG, 256), lambda i: (0, i, 0)),
        out_shape=jax.ShapeDtypeStruct((2, _NF, 256), jnp.float32),
    )(tokbig, wbig, bbig, xv)
    return out.reshape(2, _N, 16)
